# Initial kernel scaffold; baseline (speedup 1.0000x reference)
import jax
import jax.numpy as jnp
from jax.experimental import pallas as pl


def kernel(x, batch, W, b):
    raise NotImplementedError("write your pallas kernel here")


# speedup vs baseline: 1.8155x; 1.8155x over previous
import functools

import jax
import jax.numpy as jnp
from jax import lax
from jax.experimental import pallas as pl
from jax.experimental.pallas import tpu as pltpu
from jax.experimental.pallas import tpu_sc as plsc

N = 320000
D = 128
S = 1024
L = 16
NC = 2
NS = 16
NW = NC * NS
ROWS_PER_W = N // NW
CHUNK = 80
NCHUNKS = ROWS_PER_W // CHUNK
DCH = D // L


def _sc_body(x_hbm, batch_hbm, wb_hbm, out_hbm, xbuf, idxbuf, wbuf, acc):
    c = lax.axis_index("c")
    s = lax.axis_index("s")
    wid = s * NC + c
    base = wid * ROWS_PER_W

    pltpu.sync_copy(wb_hbm, wbuf)

    zeros = jnp.zeros((L,), jnp.float32)

    def _zero_row(r, _):
        for k in range(DCH):
            xbuf[r, pl.ds(k * L, L)] = zeros
        return 0

    lax.fori_loop(0, S // NS, _zero_row, 0)
    pltpu.sync_copy(xbuf.at[pl.ds(0, S // NS)], acc.at[pl.ds(s * (S // NS), S // NS)])
    plsc.subcore_barrier()

    def _chunk(t, _):
        start = base + t * CHUNK
        pltpu.sync_copy(x_hbm.at[pl.ds(start, CHUNK)], xbuf)
        pltpu.sync_copy(batch_hbm.at[pl.ds(start, CHUNK)], idxbuf)

        def _row(r, _):
            xs = [xbuf[r, pl.ds(k * L, L)] for k in range(DCH)]
            accv = xs[0] * wbuf[pl.ds(0, L)]
            for k in range(1, DCH):
                accv = accv + xs[k] * wbuf[pl.ds(k * L, L)]
            z = jnp.sum(accv)
            zv = jnp.broadcast_to(z, (L,)) + wbuf[pl.ds(D, L)]
            w = 1.0 / (1.0 + jnp.exp(-zv))
            for k in range(DCH):
                xbuf[r, pl.ds(k * L, L)] = xs[k] * w
            return 0

        lax.fori_loop(0, CHUNK, _row, 0)
        pltpu.sync_copy(xbuf, acc.at[idxbuf], add=True)
        return 0

    lax.fori_loop(0, NCHUNKS, _chunk, 0)
    plsc.subcore_barrier()

    stripe = S // NS
    pltpu.sync_copy(acc.at[pl.ds(s * stripe, stripe)],
                    out_hbm.at[c, pl.ds(s * stripe, stripe)])


@jax.jit
def _sc_weighted_segment_sum(x, batch32, wb):
    mesh = plsc.VectorSubcoreMesh(core_axis_name="c", subcore_axis_name="s")
    kern = functools.partial(
        pl.kernel,
        mesh=mesh,
        compiler_params=pltpu.CompilerParams(needs_layout_passes=False),
        out_type=jax.ShapeDtypeStruct((NC, S, D), jnp.float32),
        scratch_types=[
            pltpu.VMEM((CHUNK, D), jnp.float32),
            pltpu.VMEM((CHUNK,), jnp.int32),
            pltpu.VMEM((D + L,), jnp.float32),
            pltpu.VMEM_SHARED((S, D), jnp.float32),
        ],
    )(_sc_body)
    return kern(x, batch32, wb)


def _add_body(a_ref, o_ref):
    o_ref[...] = a_ref[0] + a_ref[1]


@jax.jit
def _tc_add(partials):
    return pl.pallas_call(
        _add_body,
        out_shape=jax.ShapeDtypeStruct((S, D), jnp.float32),
    )(partials)


def kernel(x, batch, W, b):
    batch32 = batch.astype(jnp.int32)
    wb = jnp.concatenate([W.reshape(-1), jnp.broadcast_to(b.reshape(()), (L,))])
    partials = _sc_weighted_segment_sum(x, batch32, wb)
    return _tc_add(partials)


# speedup vs baseline: 6.2175x; 3.4246x over previous
import functools

import jax
import jax.numpy as jnp
from jax import lax
from jax.experimental import pallas as pl
from jax.experimental.pallas import tpu as pltpu
from jax.experimental.pallas import tpu_sc as plsc

N = 320000
D = 128
S = 1024
L = 16
NC = 2
NS = 16
NW = NC * NS
ROWS_PER_W = N // NW
CHUNK = 80
NCHUNKS = ROWS_PER_W // CHUNK
DCH = D // L
UNROLL = 4


def _issue(x_hbm, batch_hbm, base, t, xb, ib, xsem, isem):
    @pl.when(t < NCHUNKS)
    def _():
        start = base + t * CHUNK
        pltpu.async_copy(x_hbm.at[pl.ds(start, CHUNK)], xb, xsem)
        pltpu.async_copy(batch_hbm.at[pl.ds(start, CHUNK)], ib, isem)


def _process(x_hbm, batch_hbm, base, t, xb, ib, wcs, bvec, acc, xsem, isem):
    pltpu.make_async_copy(x_hbm.at[pl.ds(0, CHUNK)], xb, xsem).wait()
    pltpu.make_async_copy(batch_hbm.at[pl.ds(0, CHUNK)], ib, isem).wait()

    def _rows(g, _):
        for u in range(UNROLL):
            r = g * UNROLL + u
            xs = [xb[r, pl.ds(k * L, L)] for k in range(DCH)]
            accv = xs[0] * wcs[0]
            for k in range(1, DCH):
                accv = accv + xs[k] * wcs[k]
            z = jnp.sum(accv)
            zv = jnp.broadcast_to(z, (L,)) + bvec
            w = 1.0 / (1.0 + jnp.exp(-zv))
            for k in range(DCH):
                xb[r, pl.ds(k * L, L)] = xs[k] * w
        return 0

    lax.fori_loop(0, CHUNK // UNROLL, _rows, 0)
    pltpu.sync_copy(xb, acc.at[ib], add=True)
    _issue(x_hbm, batch_hbm, base, t + 2, xb, ib, xsem, isem)


def _sc_body(x_hbm, batch_hbm, wb_hbm, out_hbm,
             xb0, xb1, ib0, ib1, wbuf, acc,
             xsem0, xsem1, isem0, isem1):
    c = lax.axis_index("c")
    s = lax.axis_index("s")
    wid = s * NC + c
    base = wid * ROWS_PER_W
    stripe = S // NS

    pltpu.sync_copy(wb_hbm, wbuf)

    zeros = jnp.zeros((L,), jnp.float32)

    def _zero_row(r, _):
        for k in range(DCH):
            xb0[r, pl.ds(k * L, L)] = zeros
        return 0

    lax.fori_loop(0, stripe, _zero_row, 0)
    pltpu.sync_copy(xb0.at[pl.ds(0, stripe)], acc.at[pl.ds(s * stripe, stripe)])
    plsc.subcore_barrier()

    wcs = [wbuf[pl.ds(k * L, L)] for k in range(DCH)]
    bvec = wbuf[pl.ds(D, L)]

    _issue(x_hbm, batch_hbm, base, 0, xb0, ib0, xsem0, isem0)
    _issue(x_hbm, batch_hbm, base, 1, xb1, ib1, xsem1, isem1)

    def _pair(t2, _):
        t = t2 * 2
        _process(x_hbm, batch_hbm, base, t, xb0, ib0, wcs, bvec, acc,
                 xsem0, isem0)
        _process(x_hbm, batch_hbm, base, t + 1, xb1, ib1, wcs, bvec, acc,
                 xsem1, isem1)
        return 0

    lax.fori_loop(0, NCHUNKS // 2, _pair, 0)
    _process(x_hbm, batch_hbm, base, NCHUNKS - 1, xb0, ib0, wcs, bvec, acc,
             xsem0, isem0)

    plsc.subcore_barrier()
    pltpu.sync_copy(acc.at[pl.ds(s * stripe, stripe)],
                    out_hbm.at[c, pl.ds(s * stripe, stripe)])


@jax.jit
def _sc_weighted_segment_sum(x, batch32, wb):
    mesh = plsc.VectorSubcoreMesh(core_axis_name="c", subcore_axis_name="s")
    kern = functools.partial(
        pl.kernel,
        mesh=mesh,
        compiler_params=pltpu.CompilerParams(needs_layout_passes=False),
        out_type=jax.ShapeDtypeStruct((NC, S, D), jnp.float32),
        scratch_types=[
            pltpu.VMEM((CHUNK, D), jnp.float32),
            pltpu.VMEM((CHUNK, D), jnp.float32),
            pltpu.VMEM((CHUNK,), jnp.int32),
            pltpu.VMEM((CHUNK,), jnp.int32),
            pltpu.VMEM((D + L,), jnp.float32),
            pltpu.VMEM_SHARED((S, D), jnp.float32),
            pltpu.SemaphoreType.DMA,
            pltpu.SemaphoreType.DMA,
            pltpu.SemaphoreType.DMA,
            pltpu.SemaphoreType.DMA,
        ],
    )(_sc_body)
    return kern(x, batch32, wb)


def _add_body(a_ref, o_ref):
    o_ref[...] = a_ref[0] + a_ref[1]


@jax.jit
def _tc_add(partials):
    return pl.pallas_call(
        _add_body,
        out_shape=jax.ShapeDtypeStruct((S, D), jnp.float32),
    )(partials)


def kernel(x, batch, W, b):
    batch32 = batch.astype(jnp.int32)
    wb = jnp.concatenate([W.reshape(-1), jnp.broadcast_to(b.reshape(()), (L,))])
    partials = _sc_weighted_segment_sum(x, batch32, wb)
    return _tc_add(partials)


# speedup vs baseline: 7.0519x; 1.1342x over previous
import functools

import jax
import jax.numpy as jnp
from jax import lax
from jax.experimental import pallas as pl
from jax.experimental.pallas import tpu as pltpu
from jax.experimental.pallas import tpu_sc as plsc

N = 320000
D = 128
S = 1024
L = 16
NC = 2
NS = 16
NW = NC * NS
ROWS_PER_W = N // NW
CHUNK = 80
NCHUNKS = ROWS_PER_W // CHUNK
DCH = D // L
UNROLL = 4
NBUF = 3
NLOOP = (NCHUNKS - 2) // NBUF


def _issue_fill(x_hbm, batch_hbm, base, t, buf):
    xb, ib, xsem, isem, _ = buf
    start = base + t * CHUNK
    pltpu.async_copy(x_hbm.at[pl.ds(start, CHUNK)], xb, xsem)
    pltpu.async_copy(batch_hbm.at[pl.ds(start, CHUNK)], ib, isem)


def _step(x_hbm, batch_hbm, base, t, cur, prev, wcs, bvec, acc):
    xb, ib, xsem, isem, ssem = cur
    pltpu.make_async_copy(x_hbm.at[pl.ds(0, CHUNK)], xb, xsem).wait()
    pltpu.make_async_copy(batch_hbm.at[pl.ds(0, CHUNK)], ib, isem).wait()

    def _rows(g, _):
        for u in range(UNROLL):
            r = g * UNROLL + u
            xs = [xb[r, pl.ds(k * L, L)] for k in range(DCH)]
            accv = xs[0] * wcs[0]
            for k in range(1, DCH):
                accv = accv + xs[k] * wcs[k]
            z = jnp.sum(accv)
            zv = jnp.broadcast_to(z, (L,)) + bvec
            w = 1.0 / (1.0 + jnp.exp(-zv))
            for k in range(DCH):
                xb[r, pl.ds(k * L, L)] = xs[k] * w
        return 0

    lax.fori_loop(0, CHUNK // UNROLL, _rows, 0)
    pltpu.async_copy(xb, acc.at[ib], ssem, add=True)

    @pl.when(jnp.logical_and(t >= 1, t + 2 < NCHUNKS))
    def _():
        pxb, pib, _, _, pssem = prev
        pltpu.make_async_copy(pxb, acc.at[pib], pssem).wait()
        _issue_fill(x_hbm, batch_hbm, base, t + 2, prev)


def _sc_body(x_hbm, batch_hbm, wb_hbm, out_hbm,
             xb0, xb1, xb2, ib0, ib1, ib2, wbuf, acc,
             xsem0, xsem1, xsem2, isem0, isem1, isem2,
             ssem0, ssem1, ssem2):
    c = lax.axis_index("c")
    s = lax.axis_index("s")
    wid = s * NC + c
    base = wid * ROWS_PER_W
    stripe = S // NS

    bufs = [
        (xb0, ib0, xsem0, isem0, ssem0),
        (xb1, ib1, xsem1, isem1, ssem1),
        (xb2, ib2, xsem2, isem2, ssem2),
    ]

    pltpu.sync_copy(wb_hbm, wbuf)

    zeros = jnp.zeros((L,), jnp.float32)

    def _zero_row(r, _):
        for k in range(DCH):
            xb0[r, pl.ds(k * L, L)] = zeros
        return 0

    lax.fori_loop(0, stripe, _zero_row, 0)
    pltpu.sync_copy(xb0.at[pl.ds(0, stripe)], acc.at[pl.ds(s * stripe, stripe)])
    plsc.subcore_barrier()

    wcs = [wbuf[pl.ds(k * L, L)] for k in range(DCH)]
    bvec = wbuf[pl.ds(D, L)]

    for j in range(NBUF):
        _issue_fill(x_hbm, batch_hbm, base, j, bufs[j])

    def _round(i, _):
        t = i * NBUF
        for j in range(NBUF):
            _step(x_hbm, batch_hbm, base, t + j, bufs[j], bufs[j - 1],
                  wcs, bvec, acc)
        return 0

    lax.fori_loop(0, NLOOP, _round, 0)
    _step(x_hbm, batch_hbm, base, NCHUNKS - 2, bufs[0], bufs[2],
          wcs, bvec, acc)
    _step(x_hbm, batch_hbm, base, NCHUNKS - 1, bufs[1], bufs[0],
          wcs, bvec, acc)

    for j in (2, 0, 1):
        xb, ib, _, _, ssem = bufs[j]
        pltpu.make_async_copy(xb, acc.at[ib], ssem).wait()

    plsc.subcore_barrier()
    pltpu.sync_copy(acc.at[pl.ds(s * stripe, stripe)],
                    out_hbm.at[c, pl.ds(s * stripe, stripe)])


@jax.jit
def _sc_weighted_segment_sum(x, batch32, wb):
    mesh = plsc.VectorSubcoreMesh(core_axis_name="c", subcore_axis_name="s")
    kern = functools.partial(
        pl.kernel,
        mesh=mesh,
        compiler_params=pltpu.CompilerParams(needs_layout_passes=False),
        out_type=jax.ShapeDtypeStruct((NC, S, D), jnp.float32),
        scratch_types=[
            pltpu.VMEM((CHUNK, D), jnp.float32),
            pltpu.VMEM((CHUNK, D), jnp.float32),
            pltpu.VMEM((CHUNK, D), jnp.float32),
            pltpu.VMEM((CHUNK,), jnp.int32),
            pltpu.VMEM((CHUNK,), jnp.int32),
            pltpu.VMEM((CHUNK,), jnp.int32),
            pltpu.VMEM((D + L,), jnp.float32),
            pltpu.VMEM_SHARED((S, D), jnp.float32),
            pltpu.SemaphoreType.DMA,
            pltpu.SemaphoreType.DMA,
            pltpu.SemaphoreType.DMA,
            pltpu.SemaphoreType.DMA,
            pltpu.SemaphoreType.DMA,
            pltpu.SemaphoreType.DMA,
            pltpu.SemaphoreType.DMA,
            pltpu.SemaphoreType.DMA,
            pltpu.SemaphoreType.DMA,
        ],
    )(_sc_body)
    return kern(x, batch32, wb)


def _add_body(a_ref, o_ref):
    o_ref[...] = a_ref[0] + a_ref[1]


@jax.jit
def _tc_add(partials):
    return pl.pallas_call(
        _add_body,
        out_shape=jax.ShapeDtypeStruct((S, D), jnp.float32),
    )(partials)


def kernel(x, batch, W, b):
    batch32 = batch.astype(jnp.int32)
    wb = jnp.concatenate([W.reshape(-1), jnp.broadcast_to(b.reshape(()), (L,))])
    partials = _sc_weighted_segment_sum(x, batch32, wb)
    return _tc_add(partials)


# speedup vs baseline: 7.4659x; 1.0587x over previous
import functools

import jax
import jax.numpy as jnp
from jax import lax
from jax.experimental import pallas as pl
from jax.experimental.pallas import tpu as pltpu
from jax.experimental.pallas import tpu_sc as plsc

N = 320000
D = 128
S = 1024
L = 16
NC = 2
NS = 16
NW = NC * NS
CHUNK = 80
K_SC = 65
N_SC = NW * CHUNK * K_SC
ROWS_PER_W = N_SC // NW
NCHUNKS = K_SC
DCH = D // L
UNROLL = 4
NBUF = 3
NLOOP = (NCHUNKS - 2) // NBUF

BT = 1600
N_TC = N - N_SC
NB_TC = N_TC // BT
OFF_TC = N_SC // BT


def _issue_fill(x_hbm, batch_hbm, base, t, buf):
    xb, ib, xsem, isem, _ = buf
    start = base + t * CHUNK
    pltpu.async_copy(x_hbm.at[pl.ds(start, CHUNK)], xb, xsem)
    pltpu.async_copy(batch_hbm.at[pl.ds(start, CHUNK)], ib, isem)


def _step(x_hbm, batch_hbm, base, t, cur, prev, wcs, bvec, acc):
    xb, ib, xsem, isem, ssem = cur
    pltpu.make_async_copy(x_hbm.at[pl.ds(0, CHUNK)], xb, xsem).wait()
    pltpu.make_async_copy(batch_hbm.at[pl.ds(0, CHUNK)], ib, isem).wait()

    def _rows(g, _):
        for u in range(UNROLL):
            r = g * UNROLL + u
            xs = [xb[r, pl.ds(k * L, L)] for k in range(DCH)]
            accv = xs[0] * wcs[0]
            for k in range(1, DCH):
                accv = accv + xs[k] * wcs[k]
            y = jnp.sum(accv)
            yv = jnp.broadcast_to(y, (L,)) + bvec
            w = 1.0 / (1.0 + jnp.exp(yv))
            for k in range(DCH):
                xb[r, pl.ds(k * L, L)] = xs[k] * w
        return 0

    lax.fori_loop(0, CHUNK // UNROLL, _rows, 0)
    pltpu.async_copy(xb, acc.at[ib], ssem, add=True)

    @pl.when(jnp.logical_and(t >= 1, t + 2 < NCHUNKS))
    def _():
        pxb, pib, _, _, pssem = prev
        pltpu.make_async_copy(pxb, acc.at[pib], pssem).wait()
        _issue_fill(x_hbm, batch_hbm, base, t + 2, prev)


def _sc_body(x_hbm, batch_hbm, wb_hbm, out_hbm,
             xb0, xb1, xb2, ib0, ib1, ib2, wbuf, acc,
             xsem0, xsem1, xsem2, isem0, isem1, isem2,
             ssem0, ssem1, ssem2):
    c = lax.axis_index("c")
    s = lax.axis_index("s")
    wid = s * NC + c
    base = wid * ROWS_PER_W
    stripe = S // NS

    bufs = [
        (xb0, ib0, xsem0, isem0, ssem0),
        (xb1, ib1, xsem1, isem1, ssem1),
        (xb2, ib2, xsem2, isem2, ssem2),
    ]

    pltpu.sync_copy(wb_hbm, wbuf)

    zeros = jnp.zeros((L,), jnp.float32)

    def _zero_row(r, _):
        for k in range(DCH):
            xb0[r, pl.ds(k * L, L)] = zeros
        return 0

    lax.fori_loop(0, stripe, _zero_row, 0)
    pltpu.sync_copy(xb0.at[pl.ds(0, stripe)], acc.at[pl.ds(s * stripe, stripe)])
    plsc.subcore_barrier()

    wcs = [wbuf[pl.ds(k * L, L)] for k in range(DCH)]
    bvec = wbuf[pl.ds(D, L)]

    for j in range(NBUF):
        _issue_fill(x_hbm, batch_hbm, base, j, bufs[j])

    def _round(i, _):
        t = i * NBUF
        for j in range(NBUF):
            _step(x_hbm, batch_hbm, base, t + j, bufs[j], bufs[j - 1],
                  wcs, bvec, acc)
        return 0

    lax.fori_loop(0, NLOOP, _round, 0)
    _step(x_hbm, batch_hbm, base, NCHUNKS - 2, bufs[0], bufs[2],
          wcs, bvec, acc)
    _step(x_hbm, batch_hbm, base, NCHUNKS - 1, bufs[1], bufs[0],
          wcs, bvec, acc)

    for j in (2, 0, 1):
        xb, ib, _, _, ssem = bufs[j]
        pltpu.make_async_copy(xb, acc.at[ib], ssem).wait()

    plsc.subcore_barrier()
    pltpu.sync_copy(acc.at[pl.ds(s * stripe, stripe)],
                    out_hbm.at[c, pl.ds(s * stripe, stripe)])


def _sc_weighted_segment_sum(x, batch32, wb):
    mesh = plsc.VectorSubcoreMesh(core_axis_name="c", subcore_axis_name="s")
    kern = functools.partial(
        pl.kernel,
        mesh=mesh,
        compiler_params=pltpu.CompilerParams(needs_layout_passes=False),
        out_type=jax.ShapeDtypeStruct((NC, S, D), jnp.float32),
        scratch_types=[
            pltpu.VMEM((CHUNK, D), jnp.float32),
            pltpu.VMEM((CHUNK, D), jnp.float32),
            pltpu.VMEM((CHUNK, D), jnp.float32),
            pltpu.VMEM((CHUNK,), jnp.int32),
            pltpu.VMEM((CHUNK,), jnp.int32),
            pltpu.VMEM((CHUNK,), jnp.int32),
            pltpu.VMEM((D + L,), jnp.float32),
            pltpu.VMEM_SHARED((S, D), jnp.float32),
            pltpu.SemaphoreType.DMA,
            pltpu.SemaphoreType.DMA,
            pltpu.SemaphoreType.DMA,
            pltpu.SemaphoreType.DMA,
            pltpu.SemaphoreType.DMA,
            pltpu.SemaphoreType.DMA,
            pltpu.SemaphoreType.DMA,
            pltpu.SemaphoreType.DMA,
            pltpu.SemaphoreType.DMA,
        ],
    )(_sc_body)
    return kern(x, batch32, wb)


def _tc_seg_body(x_ref, b_ref, w_ref, bias_ref, o_ref):
    i = pl.program_id(0)

    @pl.when(i == 0)
    def _():
        o_ref[...] = jnp.zeros_like(o_ref)

    xb = x_ref[...]
    z = jax.lax.dot(xb, w_ref[...],
                    preferred_element_type=jnp.float32) + bias_ref[0, 0]
    gate = jax.nn.sigmoid(z)
    p = (gate * xb).astype(jnp.bfloat16)
    seg = b_ref[0]
    oh = (lax.broadcasted_iota(jnp.int32, (S, BT), 0) == seg
          ).astype(jnp.bfloat16)
    o_ref[...] += jax.lax.dot(oh, p, preferred_element_type=jnp.float32)


def _tc_seg_sum(x, batch3d, W, bias2d):
    return pl.pallas_call(
        _tc_seg_body,
        grid=(NB_TC,),
        in_specs=[
            pl.BlockSpec((BT, D), lambda i: (i + OFF_TC, 0)),
            pl.BlockSpec((1, 1, BT), lambda i: (i + OFF_TC, 0, 0)),
            pl.BlockSpec((D, 1), lambda i: (0, 0)),
            pl.BlockSpec((1, 1), lambda i: (0, 0)),
        ],
        out_specs=pl.BlockSpec((S, D), lambda i: (0, 0)),
        out_shape=jax.ShapeDtypeStruct((S, D), jnp.float32),
    )(x, batch3d, W, bias2d)


def _add_body(a_ref, t_ref, o_ref):
    o_ref[...] = a_ref[0] + a_ref[1] + t_ref[...]


def _merge(partials, tc_part):
    return pl.pallas_call(
        _add_body,
        out_shape=jax.ShapeDtypeStruct((S, D), jnp.float32),
    )(partials, tc_part)


@jax.jit
def _run(x, batch32, wb, batch3d, W, bias2d):
    partials = _sc_weighted_segment_sum(x, batch32, wb)
    tc_part = _tc_seg_sum(x, batch3d, W, bias2d)
    return _merge(partials, tc_part)


def kernel(x, batch, W, b):
    batch32 = batch.astype(jnp.int32)
    wb = jnp.concatenate([-W.reshape(-1),
                          jnp.broadcast_to(-b.reshape(()), (L,))])
    batch3d = batch32.reshape(N // BT, 1, BT)
    bias2d = b.reshape(1, 1)
    return _run(x, batch32, wb, batch3d, W, bias2d)


# speedup vs baseline: 7.5090x; 1.0058x over previous
import functools

import jax
import jax.numpy as jnp
from jax import lax
from jax.experimental import pallas as pl
from jax.experimental.pallas import tpu as pltpu
from jax.experimental.pallas import tpu_sc as plsc

N = 320000
D = 128
S = 1024
L = 16
NC = 2
NS = 16
NW = NC * NS
CHUNK = 80
K_SC = 89
N_SC = NW * CHUNK * K_SC
ROWS_PER_W = N_SC // NW
NCHUNKS = K_SC
DCH = D // L
UNROLL = 4
NBUF = 3
NLOOP = (NCHUNKS - 2) // NBUF

BT = 640
N_TC = N - N_SC
NB_TC = N_TC // BT
OFF_TC = N_SC // BT


def _issue_fill(x_hbm, batch_hbm, base, t, buf):
    xb, ib, xsem, isem, _ = buf
    start = base + t * CHUNK
    pltpu.async_copy(x_hbm.at[pl.ds(start, CHUNK)], xb, xsem)
    pltpu.async_copy(batch_hbm.at[pl.ds(start, CHUNK)], ib, isem)


def _step(x_hbm, batch_hbm, base, t, cur, prev, wcs, bvec, acc):
    xb, ib, xsem, isem, ssem = cur
    pltpu.make_async_copy(x_hbm.at[pl.ds(0, CHUNK)], xb, xsem).wait()
    pltpu.make_async_copy(batch_hbm.at[pl.ds(0, CHUNK)], ib, isem).wait()

    def _rows(g, _):
        for u in range(UNROLL):
            r = g * UNROLL + u
            xs = [xb[r, pl.ds(k * L, L)] for k in range(DCH)]
            accv = xs[0] * wcs[0]
            for k in range(1, DCH):
                accv = accv + xs[k] * wcs[k]
            y = jnp.sum(accv)
            yv = jnp.broadcast_to(y, (L,)) + bvec
            w = 1.0 / (1.0 + jnp.exp(yv))
            for k in range(DCH):
                xb[r, pl.ds(k * L, L)] = xs[k] * w
        return 0

    lax.fori_loop(0, CHUNK // UNROLL, _rows, 0)
    pltpu.async_copy(xb, acc.at[ib], ssem, add=True)

    @pl.when(jnp.logical_and(t >= 1, t + 2 < NCHUNKS))
    def _():
        pxb, pib, _, _, pssem = prev
        pltpu.make_async_copy(pxb, acc.at[pib], pssem).wait()
        _issue_fill(x_hbm, batch_hbm, base, t + 2, prev)


def _sc_body(x_hbm, batch_hbm, wb_hbm, out_hbm,
             xb0, xb1, xb2, ib0, ib1, ib2, wbuf, acc,
             xsem0, xsem1, xsem2, isem0, isem1, isem2,
             ssem0, ssem1, ssem2):
    c = lax.axis_index("c")
    s = lax.axis_index("s")
    wid = s * NC + c
    base = wid * ROWS_PER_W
    stripe = S // NS

    bufs = [
        (xb0, ib0, xsem0, isem0, ssem0),
        (xb1, ib1, xsem1, isem1, ssem1),
        (xb2, ib2, xsem2, isem2, ssem2),
    ]

    pltpu.sync_copy(wb_hbm, wbuf)

    zeros = jnp.zeros((L,), jnp.float32)

    def _zero_row(r, _):
        for k in range(DCH):
            xb0[r, pl.ds(k * L, L)] = zeros
        return 0

    lax.fori_loop(0, stripe, _zero_row, 0)
    pltpu.sync_copy(xb0.at[pl.ds(0, stripe)], acc.at[pl.ds(s * stripe, stripe)])
    plsc.subcore_barrier()

    wcs = [wbuf[pl.ds(k * L, L)] for k in range(DCH)]
    bvec = wbuf[pl.ds(D, L)]

    for j in range(NBUF):
        _issue_fill(x_hbm, batch_hbm, base, j, bufs[j])

    def _round(i, _):
        t = i * NBUF
        for j in range(NBUF):
            _step(x_hbm, batch_hbm, base, t + j, bufs[j], bufs[j - 1],
                  wcs, bvec, acc)
        return 0

    lax.fori_loop(0, NLOOP, _round, 0)
    _step(x_hbm, batch_hbm, base, NCHUNKS - 2, bufs[0], bufs[2],
          wcs, bvec, acc)
    _step(x_hbm, batch_hbm, base, NCHUNKS - 1, bufs[1], bufs[0],
          wcs, bvec, acc)

    for j in (2, 0, 1):
        xb, ib, _, _, ssem = bufs[j]
        pltpu.make_async_copy(xb, acc.at[ib], ssem).wait()

    plsc.subcore_barrier()
    pltpu.sync_copy(acc.at[pl.ds(s * stripe, stripe)],
                    out_hbm.at[c, pl.ds(s * stripe, stripe)])


def _sc_weighted_segment_sum(x, batch32, wb):
    mesh = plsc.VectorSubcoreMesh(core_axis_name="c", subcore_axis_name="s")
    kern = functools.partial(
        pl.kernel,
        mesh=mesh,
        compiler_params=pltpu.CompilerParams(needs_layout_passes=False),
        out_type=jax.ShapeDtypeStruct((NC, S, D), jnp.float32),
        scratch_types=[
            pltpu.VMEM((CHUNK, D), jnp.float32),
            pltpu.VMEM((CHUNK, D), jnp.float32),
            pltpu.VMEM((CHUNK, D), jnp.float32),
            pltpu.VMEM((CHUNK,), jnp.int32),
            pltpu.VMEM((CHUNK,), jnp.int32),
            pltpu.VMEM((CHUNK,), jnp.int32),
            pltpu.VMEM((D + L,), jnp.float32),
            pltpu.VMEM_SHARED((S, D), jnp.float32),
            pltpu.SemaphoreType.DMA,
            pltpu.SemaphoreType.DMA,
            pltpu.SemaphoreType.DMA,
            pltpu.SemaphoreType.DMA,
            pltpu.SemaphoreType.DMA,
            pltpu.SemaphoreType.DMA,
            pltpu.SemaphoreType.DMA,
            pltpu.SemaphoreType.DMA,
            pltpu.SemaphoreType.DMA,
        ],
    )(_sc_body)
    return kern(x, batch32, wb)


def _tc_seg_body(x_ref, b_ref, w_ref, bias_ref, o_ref):
    i = pl.program_id(0)

    @pl.when(i == 0)
    def _():
        o_ref[...] = jnp.zeros_like(o_ref)

    xb = x_ref[...]
    z = jax.lax.dot(xb, w_ref[...],
                    preferred_element_type=jnp.float32) + bias_ref[0, 0]
    gate = jax.nn.sigmoid(z)
    p = (gate * xb).astype(jnp.bfloat16)
    seg = b_ref[0]
    oh = (lax.broadcasted_iota(jnp.int16, (S, BT), 0) == seg
          ).astype(jnp.bfloat16)
    o_ref[...] += jax.lax.dot(oh, p, preferred_element_type=jnp.float32)


def _tc_seg_sum(x, batch3d, W, bias2d):
    return pl.pallas_call(
        _tc_seg_body,
        grid=(NB_TC,),
        in_specs=[
            pl.BlockSpec((BT, D), lambda i: (i + OFF_TC, 0)),
            pl.BlockSpec((1, 1, BT), lambda i: (i + OFF_TC, 0, 0)),
            pl.BlockSpec((D, 1), lambda i: (0, 0)),
            pl.BlockSpec((1, 1), lambda i: (0, 0)),
        ],
        out_specs=pl.BlockSpec((S, D), lambda i: (0, 0)),
        out_shape=jax.ShapeDtypeStruct((S, D), jnp.float32),
    )(x, batch3d, W, bias2d)


def _add_body(a_ref, t_ref, o_ref):
    o_ref[...] = a_ref[0] + a_ref[1] + t_ref[...]


def _merge(partials, tc_part):
    return pl.pallas_call(
        _add_body,
        out_shape=jax.ShapeDtypeStruct((S, D), jnp.float32),
    )(partials, tc_part)


@jax.jit
def _run(x, batch32, wb, batch3d, W, bias2d):
    partials = _sc_weighted_segment_sum(x, batch32, wb)
    tc_part = _tc_seg_sum(x, batch3d, W, bias2d)
    return _merge(partials, tc_part)


def kernel(x, batch, W, b):
    batch32 = batch.astype(jnp.int32)
    wb = jnp.concatenate([-W.reshape(-1),
                          jnp.broadcast_to(-b.reshape(()), (L,))])
    batch3d = batch32.astype(jnp.int16).reshape(N // BT, 1, BT)
    bias2d = b.reshape(1, 1)
    return _run(x, batch32, wb, batch3d, W, bias2d)


# speedup vs baseline: 9.3744x; 1.2484x over previous
import functools

import jax
import jax.numpy as jnp
from jax import lax
from jax.experimental import pallas as pl
from jax.experimental.pallas import tpu as pltpu
from jax.experimental.pallas import tpu_sc as plsc

N = 320000
D = 128
S = 1024
L = 16
NC = 2
NS = 16
NW = NC * NS
CHUNK = 80
K_SC = 80
N_SC = NW * CHUNK * K_SC
ROWS_PER_W = N_SC // NW
NCHUNKS = K_SC
DCH = D // L
UNROLL = 4
NBUF = 3
NLOOP = (NCHUNKS - 2) // NBUF

BT = 2560
N_TC = N - N_SC
NB_TC = N_TC // BT
OFF_TC = N_SC // BT


def _issue_fill(x_hbm, batch_hbm, base, t, buf):
    xb, ib, xsem, isem, _ = buf
    start = base + t * CHUNK
    pltpu.async_copy(x_hbm.at[pl.ds(start, CHUNK)], xb, xsem)
    pltpu.async_copy(batch_hbm.at[pl.ds(start, CHUNK)], ib, isem)


def _step(x_hbm, batch_hbm, base, t, cur, prev, wcs, bvec, acc):
    xb, ib, xsem, isem, ssem = cur
    pltpu.make_async_copy(x_hbm.at[pl.ds(0, CHUNK)], xb, xsem).wait()
    pltpu.make_async_copy(batch_hbm.at[pl.ds(0, CHUNK)], ib, isem).wait()

    def _rows(g, _):
        for u in range(UNROLL):
            r = g * UNROLL + u
            xs = [xb[r, pl.ds(k * L, L)] for k in range(DCH)]
            accv = xs[0] * wcs[0]
            for k in range(1, DCH):
                accv = accv + xs[k] * wcs[k]
            y = jnp.sum(accv)
            yv = jnp.broadcast_to(y, (L,)) + bvec
            w = 1.0 / (1.0 + jnp.exp(yv))
            for k in range(DCH):
                xb[r, pl.ds(k * L, L)] = xs[k] * w
        return 0

    lax.fori_loop(0, CHUNK // UNROLL, _rows, 0)
    pltpu.async_copy(xb, acc.at[ib], ssem, add=True)

    @pl.when(jnp.logical_and(t >= 1, t + 2 < NCHUNKS))
    def _():
        pxb, pib, _, _, pssem = prev
        pltpu.make_async_copy(pxb, acc.at[pib], pssem).wait()
        _issue_fill(x_hbm, batch_hbm, base, t + 2, prev)


def _sc_body(x_hbm, batch_hbm, wb_hbm, out_hbm,
             xb0, xb1, xb2, ib0, ib1, ib2, wbuf, acc,
             xsem0, xsem1, xsem2, isem0, isem1, isem2,
             ssem0, ssem1, ssem2):
    c = lax.axis_index("c")
    s = lax.axis_index("s")
    wid = s * NC + c
    base = wid * ROWS_PER_W
    stripe = S // NS

    bufs = [
        (xb0, ib0, xsem0, isem0, ssem0),
        (xb1, ib1, xsem1, isem1, ssem1),
        (xb2, ib2, xsem2, isem2, ssem2),
    ]

    pltpu.sync_copy(wb_hbm, wbuf)

    zeros = jnp.zeros((L,), jnp.float32)

    def _zero_row(r, _):
        for k in range(DCH):
            xb0[r, pl.ds(k * L, L)] = zeros
        return 0

    lax.fori_loop(0, stripe, _zero_row, 0)
    pltpu.sync_copy(xb0.at[pl.ds(0, stripe)], acc.at[pl.ds(s * stripe, stripe)])
    plsc.subcore_barrier()

    wcs = [wbuf[pl.ds(k * L, L)] for k in range(DCH)]
    bvec = wbuf[pl.ds(D, L)]

    for j in range(NBUF):
        _issue_fill(x_hbm, batch_hbm, base, j, bufs[j])

    def _round(i, _):
        t = i * NBUF
        for j in range(NBUF):
            _step(x_hbm, batch_hbm, base, t + j, bufs[j], bufs[j - 1],
                  wcs, bvec, acc)
        return 0

    lax.fori_loop(0, NLOOP, _round, 0)
    _step(x_hbm, batch_hbm, base, NCHUNKS - 2, bufs[0], bufs[2],
          wcs, bvec, acc)
    _step(x_hbm, batch_hbm, base, NCHUNKS - 1, bufs[1], bufs[0],
          wcs, bvec, acc)

    for j in (2, 0, 1):
        xb, ib, _, _, ssem = bufs[j]
        pltpu.make_async_copy(xb, acc.at[ib], ssem).wait()

    plsc.subcore_barrier()
    pltpu.sync_copy(acc.at[pl.ds(s * stripe, stripe)],
                    out_hbm.at[c, pl.ds(s * stripe, stripe)])


def _sc_weighted_segment_sum(x, batch32, wb):
    mesh = plsc.VectorSubcoreMesh(core_axis_name="c", subcore_axis_name="s")
    kern = functools.partial(
        pl.kernel,
        mesh=mesh,
        compiler_params=pltpu.CompilerParams(needs_layout_passes=False),
        out_type=jax.ShapeDtypeStruct((NC, S, D), jnp.float32),
        scratch_types=[
            pltpu.VMEM((CHUNK, D), jnp.float32),
            pltpu.VMEM((CHUNK, D), jnp.float32),
            pltpu.VMEM((CHUNK, D), jnp.float32),
            pltpu.VMEM((CHUNK,), jnp.int32),
            pltpu.VMEM((CHUNK,), jnp.int32),
            pltpu.VMEM((CHUNK,), jnp.int32),
            pltpu.VMEM((D + L,), jnp.float32),
            pltpu.VMEM_SHARED((S, D), jnp.float32),
            pltpu.SemaphoreType.DMA,
            pltpu.SemaphoreType.DMA,
            pltpu.SemaphoreType.DMA,
            pltpu.SemaphoreType.DMA,
            pltpu.SemaphoreType.DMA,
            pltpu.SemaphoreType.DMA,
            pltpu.SemaphoreType.DMA,
            pltpu.SemaphoreType.DMA,
            pltpu.SemaphoreType.DMA,
        ],
    )(_sc_body)
    return kern(x, batch32, wb)


def _tc_seg_body(x_ref, b_ref, w_ref, bias_ref, o_ref):
    i = pl.program_id(0)

    @pl.when(i == 0)
    def _():
        o_ref[...] = jnp.zeros_like(o_ref)

    xb = x_ref[...]
    z = jax.lax.dot(xb, w_ref[...],
                    preferred_element_type=jnp.float32) + bias_ref[0, 0]
    gate = jax.nn.sigmoid(z)
    p = (gate * xb).astype(jnp.bfloat16)
    seg = b_ref[0]
    oh = (lax.broadcasted_iota(jnp.int16, (S, BT), 0) == seg
          ).astype(jnp.bfloat16)
    o_ref[...] += jax.lax.dot(oh, p, preferred_element_type=jnp.float32)


def _tc_seg_sum(x, batch3d, W, bias2d):
    return pl.pallas_call(
        _tc_seg_body,
        grid=(NB_TC,),
        in_specs=[
            pl.BlockSpec((BT, D), lambda i: (i + OFF_TC, 0)),
            pl.BlockSpec((1, 1, BT), lambda i: (i + OFF_TC, 0, 0)),
            pl.BlockSpec((D, 1), lambda i: (0, 0)),
            pl.BlockSpec((1, 1), lambda i: (0, 0)),
        ],
        out_specs=pl.BlockSpec((S, D), lambda i: (0, 0)),
        out_shape=jax.ShapeDtypeStruct((S, D), jnp.float32),
    )(x, batch3d, W, bias2d)


def _add_body(a_ref, t_ref, o_ref):
    o_ref[...] = a_ref[0] + a_ref[1] + t_ref[...]


def _merge(partials, tc_part):
    return pl.pallas_call(
        _add_body,
        out_shape=jax.ShapeDtypeStruct((S, D), jnp.float32),
    )(partials, tc_part)


@jax.jit
def _run(x, batch32, wb, batch3d, W, bias2d):
    partials = _sc_weighted_segment_sum(x, batch32, wb)
    tc_part = _tc_seg_sum(x, batch3d, W, bias2d)
    return _merge(partials, tc_part)


def kernel(x, batch, W, b):
    batch32 = batch.astype(jnp.int32)
    wb = jnp.concatenate([-W.reshape(-1),
                          jnp.broadcast_to(-b.reshape(()), (L,))])
    batch3d = batch32.astype(jnp.int16).reshape(N // BT, 1, BT)
    bias2d = b.reshape(1, 1)
    return _run(x, batch32, wb, batch3d, W, bias2d)
